# Initial kernel scaffold; baseline (speedup 1.0000x reference)
import functools

import numpy as np
import jax
import jax.numpy as jnp
from jax import lax
from jax.experimental import pallas as pl
from jax.experimental.pallas import tpu as pltpu
from jax.experimental.pallas import tpu_sc as plsc

N_NODES = 10000
N_EDGES = 320000
D_IN = 128
D_SCAL = 16
D_HID = 64
NUM_NEIGHBORS = 32.0


def _silu_norm_const():
    x = np.linspace(-12.0, 12.0, 200001)
    pdf = np.exp(-0.5 * x * x) / np.sqrt(2.0 * np.pi)
    f = x / (1.0 + np.exp(-x))
    m2 = np.trapz(f * f * pdf, x)
    return 1.0 / np.sqrt(m2)


_SILU_NORM = float(_silu_norm_const())

_EBLK = 3200


def _edge_w_body(es_ref, ea_ref, w1_ref, w2_ref, out_ref):
    es = es_ref[...]
    h = jnp.dot(es, w1_ref[...] * (1.0 / np.sqrt(D_SCAL)),
                preferred_element_type=jnp.float32)
    h = (h / (1.0 + jnp.exp(-h))) * _SILU_NORM
    w = jnp.dot(h, w2_ref[...] * (1.0 / np.sqrt(D_HID)),
                preferred_element_type=jnp.float32)
    out_ref[...] = w * ea_ref[...]


def _edge_weights(edge_scalars, edge_attr, fc_w1, fc_w2):
    grid = (N_EDGES // _EBLK,)
    return pl.pallas_call(
        _edge_w_body,
        grid=grid,
        in_specs=[
            pl.BlockSpec((_EBLK, D_SCAL), lambda i: (i, 0)),
            pl.BlockSpec((_EBLK, 1), lambda i: (i, 0)),
            pl.BlockSpec((D_SCAL, D_HID), lambda i: (0, 0)),
            pl.BlockSpec((D_HID, D_IN), lambda i: (0, 0)),
        ],
        out_specs=pl.BlockSpec((_EBLK, D_IN), lambda i: (i, 0)),
        out_shape=jax.ShapeDtypeStruct((N_EDGES, D_IN), jnp.float32),
    )(edge_scalars, edge_attr, fc_w1, fc_w2)


_NBLK = 2000


def _node_pre_body(x_ref, attr_ref, lin1_ref, scw_ref, nf_ref, nsc_ref):
    x = x_ref[...]
    s = attr_ref[...] * (1.0 / np.sqrt(D_IN))
    nf_ref[...] = jnp.dot(x, lin1_ref[...], preferred_element_type=jnp.float32) * s
    nsc_ref[...] = jnp.dot(x, scw_ref[...], preferred_element_type=jnp.float32) * s


def _node_pre(node_input, node_attr, lin1, scw):
    grid = (N_NODES // _NBLK,)
    return pl.pallas_call(
        _node_pre_body,
        grid=grid,
        in_specs=[
            pl.BlockSpec((_NBLK, D_IN), lambda i: (i, 0)),
            pl.BlockSpec((_NBLK, 1), lambda i: (i, 0)),
            pl.BlockSpec((D_IN, D_IN), lambda i: (0, 0)),
            pl.BlockSpec((D_IN, D_IN), lambda i: (0, 0)),
        ],
        out_specs=[
            pl.BlockSpec((_NBLK, D_IN), lambda i: (i, 0)),
            pl.BlockSpec((_NBLK, D_IN), lambda i: (i, 0)),
        ],
        out_shape=[
            jax.ShapeDtypeStruct((N_NODES, D_IN), jnp.float32),
            jax.ShapeDtypeStruct((N_NODES, D_IN), jnp.float32),
        ],
    )(node_input, node_attr, lin1, scw)


_NWORK = 32
_EPW = N_EDGES // _NWORK
_CB = 80
_NCH = _EPW // _CB
_ZROWS = 125
_SLAB = N_NODES // 16


def _sc_body(nf_hbm, wea_hbm, src_hbm, dst_hbm, out0, out1,
             idx_v, dst_v, rows_v, w_v, z_v, acc_sh, sem):
    c = lax.axis_index("c")
    s = lax.axis_index("s")
    wid = s * 2 + c

    zero16 = jnp.zeros((16,), jnp.float32)

    def zbody(i, carry):
        z_v[i // 8, pl.ds((i % 8) * 16, 16)] = zero16
        return carry

    lax.fori_loop(0, _ZROWS * 8, zbody, 0, unroll=8)

    slab = s * _SLAB

    def zslab(k, carry):
        pltpu.sync_copy(z_v, acc_sh.at[pl.ds(slab + k * _ZROWS, _ZROWS), :])
        return carry

    lax.fori_loop(0, 5, zslab, 0)
    plsc.subcore_barrier()

    base0 = wid * _EPW

    def chunk(j, carry):
        base = base0 + j * _CB
        pltpu.sync_copy(src_hbm.at[pl.ds(base, _CB)], idx_v)
        pltpu.sync_copy(dst_hbm.at[pl.ds(base, _CB)], dst_v)
        pltpu.async_copy(nf_hbm.at[idx_v], rows_v, sem).wait()
        pltpu.sync_copy(wea_hbm.at[pl.ds(base, _CB), :], w_v)

        def mul(i, cc):
            r = i // 8
            col = (i % 8) * 16
            rows_v[r, pl.ds(col, 16)] = (
                rows_v[r, pl.ds(col, 16)] * w_v[r, pl.ds(col, 16)])
            return cc

        lax.fori_loop(0, _CB * 8, mul, 0, unroll=8)
        pltpu.sync_copy(rows_v, acc_sh.at[dst_v], add=True)
        return carry

    lax.fori_loop(0, _NCH, chunk, 0)
    plsc.subcore_barrier()

    @pl.when(c == 0)
    def _():
        pltpu.sync_copy(acc_sh.at[pl.ds(slab, _SLAB), :],
                        out0.at[pl.ds(slab, _SLAB), :])

    @pl.when(c == 1)
    def _():
        pltpu.sync_copy(acc_sh.at[pl.ds(slab, _SLAB), :],
                        out1.at[pl.ds(slab, _SLAB), :])


_sc_scatter = functools.partial(
    pl.kernel,
    out_type=(
        jax.ShapeDtypeStruct((N_NODES, D_IN), jnp.float32),
        jax.ShapeDtypeStruct((N_NODES, D_IN), jnp.float32),
    ),
    mesh=plsc.VectorSubcoreMesh(core_axis_name="c", subcore_axis_name="s"),
    scratch_types=[
        pltpu.VMEM((_CB,), jnp.int32),
        pltpu.VMEM((_CB,), jnp.int32),
        pltpu.VMEM((_CB, D_IN), jnp.float32),
        pltpu.VMEM((_CB, D_IN), jnp.float32),
        pltpu.VMEM((_ZROWS, D_IN), jnp.float32),
        pltpu.VMEM_SHARED((N_NODES, D_IN), jnp.float32),
        pltpu.SemaphoreType.DMA,
    ],
)(_sc_body)


def _final_body(a0_ref, a1_ref, attr_ref, lin2_ref, lin3_ref, nsc_ref, out_ref):
    agg = (a0_ref[...] + a1_ref[...]) * (1.0 / np.sqrt(NUM_NEIGHBORS))
    a = attr_ref[...] * (1.0 / np.sqrt(D_IN))
    conv = jnp.dot(agg, lin2_ref[...], preferred_element_type=jnp.float32) * a
    ang = 0.1 * jnp.dot(agg, lin3_ref[...], preferred_element_type=jnp.float32) * a
    out_ref[...] = jnp.cos(ang) * nsc_ref[...] + jnp.sin(ang) * conv


def _final(a0, a1, node_attr, lin2, lin3, nsc):
    grid = (N_NODES // _NBLK,)
    return pl.pallas_call(
        _final_body,
        grid=grid,
        in_specs=[
            pl.BlockSpec((_NBLK, D_IN), lambda i: (i, 0)),
            pl.BlockSpec((_NBLK, D_IN), lambda i: (i, 0)),
            pl.BlockSpec((_NBLK, 1), lambda i: (i, 0)),
            pl.BlockSpec((D_IN, D_IN), lambda i: (0, 0)),
            pl.BlockSpec((D_IN, 1), lambda i: (0, 0)),
            pl.BlockSpec((_NBLK, D_IN), lambda i: (i, 0)),
        ],
        out_specs=pl.BlockSpec((_NBLK, D_IN), lambda i: (i, 0)),
        out_shape=jax.ShapeDtypeStruct((N_NODES, D_IN), jnp.float32),
    )(a0, a1, node_attr, lin2, lin3, nsc)


def kernel(node_input, node_attr, edge_src, edge_dst, edge_attr, edge_scalars,
           sc_w, lin1_w, fc_w1, fc_w2, lin2_w, lin3_w):
    src = edge_src.astype(jnp.int32)
    dst = edge_dst.astype(jnp.int32)
    wea = _edge_weights(edge_scalars, edge_attr, fc_w1, fc_w2)
    nf, nsc = _node_pre(node_input, node_attr, lin1_w[:, 0, :], sc_w[:, 0, :])
    a0, a1 = _sc_scatter(nf, wea, src, dst)
    return _final(a0, a1, node_attr, lin2_w[:, 0, :], lin3_w[:, 0, :], nsc)


# speedup vs baseline: 1.6273x; 1.6273x over previous
import functools

import numpy as np
import jax
import jax.numpy as jnp
from jax import lax
from jax.experimental import pallas as pl
from jax.experimental.pallas import tpu as pltpu
from jax.experimental.pallas import tpu_sc as plsc

N_NODES = 10000
N_EDGES = 320000
D_IN = 128
D_SCAL = 16
D_HID = 64
NUM_NEIGHBORS = 32.0


def _silu_norm_const():
    x = np.linspace(-12.0, 12.0, 200001)
    pdf = np.exp(-0.5 * x * x) / np.sqrt(2.0 * np.pi)
    f = x / (1.0 + np.exp(-x))
    m2 = np.trapz(f * f * pdf, x)
    return 1.0 / np.sqrt(m2)


_SILU_NORM = float(_silu_norm_const())

_EBLK = 3200


def _edge_w_body(es_ref, ea_ref, w1_ref, w2_ref, out_ref):
    es = es_ref[...]
    h = jnp.dot(es, w1_ref[...] * (1.0 / np.sqrt(D_SCAL)),
                preferred_element_type=jnp.float32)
    h = (h / (1.0 + jnp.exp(-h))) * _SILU_NORM
    w = jnp.dot(h, w2_ref[...] * (1.0 / np.sqrt(D_HID)),
                preferred_element_type=jnp.float32)
    out_ref[...] = w * ea_ref[...]


def _edge_weights(edge_scalars, edge_attr, fc_w1, fc_w2):
    grid = (N_EDGES // _EBLK,)
    return pl.pallas_call(
        _edge_w_body,
        grid=grid,
        in_specs=[
            pl.BlockSpec((_EBLK, D_SCAL), lambda i: (i, 0)),
            pl.BlockSpec((_EBLK, 1), lambda i: (i, 0)),
            pl.BlockSpec((D_SCAL, D_HID), lambda i: (0, 0)),
            pl.BlockSpec((D_HID, D_IN), lambda i: (0, 0)),
        ],
        out_specs=pl.BlockSpec((_EBLK, D_IN), lambda i: (i, 0)),
        out_shape=jax.ShapeDtypeStruct((N_EDGES, D_IN), jnp.float32),
    )(edge_scalars, edge_attr, fc_w1, fc_w2)


_NBLK = 2000


def _node_pre_body(x_ref, attr_ref, lin1_ref, scw_ref, nf_ref, nsc_ref):
    x = x_ref[...]
    s = attr_ref[...] * (1.0 / np.sqrt(D_IN))
    nf_ref[...] = jnp.dot(x, lin1_ref[...], preferred_element_type=jnp.float32) * s
    nsc_ref[...] = jnp.dot(x, scw_ref[...], preferred_element_type=jnp.float32) * s


def _node_pre(node_input, node_attr, lin1, scw):
    grid = (N_NODES // _NBLK,)
    return pl.pallas_call(
        _node_pre_body,
        grid=grid,
        in_specs=[
            pl.BlockSpec((_NBLK, D_IN), lambda i: (i, 0)),
            pl.BlockSpec((_NBLK, 1), lambda i: (i, 0)),
            pl.BlockSpec((D_IN, D_IN), lambda i: (0, 0)),
            pl.BlockSpec((D_IN, D_IN), lambda i: (0, 0)),
        ],
        out_specs=[
            pl.BlockSpec((_NBLK, D_IN), lambda i: (i, 0)),
            pl.BlockSpec((_NBLK, D_IN), lambda i: (i, 0)),
        ],
        out_shape=[
            jax.ShapeDtypeStruct((N_NODES, D_IN), jnp.float32),
            jax.ShapeDtypeStruct((N_NODES, D_IN), jnp.float32),
        ],
    )(node_input, node_attr, lin1, scw)


_NWORK = 32
_EPW = N_EDGES // _NWORK
_CB = 80
_NCH = _EPW // _CB
_ZROWS = 208
_SLAB = 624
_TAIL = N_NODES - 16 * _SLAB


def _sc_body(nf_hbm, wea_hbm, src_hbm, dst_hbm, out0, out1,
             idx_v, dst_v, rows_v, w_v, z_v, acc_sh, sem):
    c = lax.axis_index("c")
    s = lax.axis_index("s")
    wid = s * 2 + c

    zero16 = jnp.zeros((16,), jnp.float32)

    def zbody(i, carry):
        z_v[i // 8, pl.ds((i % 8) * 16, 16)] = zero16
        return carry

    lax.fori_loop(0, _ZROWS * 8, zbody, 0, unroll=8)

    slab = s * _SLAB

    def zslab(k, carry):
        pltpu.sync_copy(z_v, acc_sh.at[pl.ds(slab + k * _ZROWS, _ZROWS), :])
        return carry

    lax.fori_loop(0, _SLAB // _ZROWS, zslab, 0)

    @pl.when(s == 0)
    def _():
        pltpu.sync_copy(z_v.at[pl.ds(0, _TAIL), :],
                        acc_sh.at[pl.ds(16 * _SLAB, _TAIL), :])

    plsc.subcore_barrier()

    base0 = wid * _EPW

    def chunk(j, carry):
        base = base0 + j * _CB
        pltpu.sync_copy(src_hbm.at[pl.ds(base, _CB)], idx_v)
        pltpu.sync_copy(dst_hbm.at[pl.ds(base, _CB)], dst_v)
        pltpu.async_copy(nf_hbm.at[idx_v], rows_v, sem).wait()
        pltpu.sync_copy(wea_hbm.at[pl.ds(base, _CB), :], w_v)

        def mul(i, cc):
            r = i // 8
            col = (i % 8) * 16
            rows_v[r, pl.ds(col, 16)] = (
                rows_v[r, pl.ds(col, 16)] * w_v[r, pl.ds(col, 16)])
            return cc

        lax.fori_loop(0, _CB * 8, mul, 0, unroll=8)
        pltpu.sync_copy(rows_v, acc_sh.at[dst_v], add=True)
        return carry

    lax.fori_loop(0, _NCH, chunk, 0)
    plsc.subcore_barrier()

    @pl.when(c == 0)
    def _():
        pltpu.sync_copy(acc_sh.at[pl.ds(slab, _SLAB), :],
                        out0.at[pl.ds(slab, _SLAB), :])

    @pl.when(c == 1)
    def _():
        pltpu.sync_copy(acc_sh.at[pl.ds(slab, _SLAB), :],
                        out1.at[pl.ds(slab, _SLAB), :])

    @pl.when((s == 0) & (c == 0))
    def _():
        pltpu.sync_copy(acc_sh.at[pl.ds(16 * _SLAB, _TAIL), :],
                        out0.at[pl.ds(16 * _SLAB, _TAIL), :])

    @pl.when((s == 0) & (c == 1))
    def _():
        pltpu.sync_copy(acc_sh.at[pl.ds(16 * _SLAB, _TAIL), :],
                        out1.at[pl.ds(16 * _SLAB, _TAIL), :])


def _sc_scatter(nf, wea, src, dst):
    call = functools.partial(
        pl.kernel,
        out_type=(
            jax.ShapeDtypeStruct((N_NODES, D_IN), jnp.float32),
            jax.ShapeDtypeStruct((N_NODES, D_IN), jnp.float32),
        ),
        mesh=plsc.VectorSubcoreMesh(core_axis_name="c", subcore_axis_name="s",
                                    num_cores=2, num_subcores=16),
        scratch_types=[
            pltpu.VMEM((_CB,), jnp.int32),
            pltpu.VMEM((_CB,), jnp.int32),
            pltpu.VMEM((_CB, D_IN), jnp.float32),
            pltpu.VMEM((_CB, D_IN), jnp.float32),
            pltpu.VMEM((_ZROWS, D_IN), jnp.float32),
            pltpu.VMEM_SHARED((N_NODES, D_IN), jnp.float32),
            pltpu.SemaphoreType.DMA,
        ],
    )(_sc_body)
    return call(nf, wea, src, dst)


def _final_body(a0_ref, a1_ref, attr_ref, lin2_ref, lin3_ref, nsc_ref, out_ref):
    agg = (a0_ref[...] + a1_ref[...]) * (1.0 / np.sqrt(NUM_NEIGHBORS))
    a = attr_ref[...] * (1.0 / np.sqrt(D_IN))
    conv = jnp.dot(agg, lin2_ref[...], preferred_element_type=jnp.float32) * a
    ang = 0.1 * jnp.dot(agg, lin3_ref[...], preferred_element_type=jnp.float32) * a
    out_ref[...] = jnp.cos(ang) * nsc_ref[...] + jnp.sin(ang) * conv


def _final(a0, a1, node_attr, lin2, lin3, nsc):
    grid = (N_NODES // _NBLK,)
    return pl.pallas_call(
        _final_body,
        grid=grid,
        in_specs=[
            pl.BlockSpec((_NBLK, D_IN), lambda i: (i, 0)),
            pl.BlockSpec((_NBLK, D_IN), lambda i: (i, 0)),
            pl.BlockSpec((_NBLK, 1), lambda i: (i, 0)),
            pl.BlockSpec((D_IN, D_IN), lambda i: (0, 0)),
            pl.BlockSpec((D_IN, 1), lambda i: (0, 0)),
            pl.BlockSpec((_NBLK, D_IN), lambda i: (i, 0)),
        ],
        out_specs=pl.BlockSpec((_NBLK, D_IN), lambda i: (i, 0)),
        out_shape=jax.ShapeDtypeStruct((N_NODES, D_IN), jnp.float32),
    )(a0, a1, node_attr, lin2, lin3, nsc)


def kernel(node_input, node_attr, edge_src, edge_dst, edge_attr, edge_scalars,
           sc_w, lin1_w, fc_w1, fc_w2, lin2_w, lin3_w):
    src = edge_src.astype(jnp.int32)
    dst = edge_dst.astype(jnp.int32)
    wea = _edge_weights(edge_scalars, edge_attr, fc_w1, fc_w2)
    nf, nsc = _node_pre(node_input, node_attr, lin1_w[:, 0, :], sc_w[:, 0, :])
    a0, a1 = _sc_scatter(nf, wea, src, dst)
    return _final(a0, a1, node_attr, lin2_w[:, 0, :], lin3_w[:, 0, :], nsc)


# speedup vs baseline: 2.3408x; 1.4385x over previous
import functools

import numpy as np
import jax
import jax.numpy as jnp
from jax import lax
from jax.experimental import pallas as pl
from jax.experimental.pallas import tpu as pltpu
from jax.experimental.pallas import tpu_sc as plsc

N_NODES = 10000
N_EDGES = 320000
D_IN = 128
D_SCAL = 16
D_HID = 64
NUM_NEIGHBORS = 32.0


def _silu_norm_const():
    x = np.linspace(-12.0, 12.0, 200001)
    pdf = np.exp(-0.5 * x * x) / np.sqrt(2.0 * np.pi)
    f = x / (1.0 + np.exp(-x))
    m2 = np.trapz(f * f * pdf, x)
    return 1.0 / np.sqrt(m2)


_SILU_NORM = float(_silu_norm_const())

_EBLK = 3200


def _edge_w_body(es_ref, ea_ref, w1_ref, w2_ref, out_ref):
    es = es_ref[...]
    h = jnp.dot(es, w1_ref[...] * (1.0 / np.sqrt(D_SCAL)),
                preferred_element_type=jnp.float32)
    h = (h / (1.0 + jnp.exp(-h))) * _SILU_NORM
    w = jnp.dot(h, w2_ref[...] * (1.0 / np.sqrt(D_HID)),
                preferred_element_type=jnp.float32)
    out_ref[...] = w * ea_ref[...]


def _edge_weights(edge_scalars, edge_attr, fc_w1, fc_w2):
    grid = (N_EDGES // _EBLK,)
    return pl.pallas_call(
        _edge_w_body,
        grid=grid,
        in_specs=[
            pl.BlockSpec((_EBLK, D_SCAL), lambda i: (i, 0)),
            pl.BlockSpec((_EBLK, 1), lambda i: (i, 0)),
            pl.BlockSpec((D_SCAL, D_HID), lambda i: (0, 0)),
            pl.BlockSpec((D_HID, D_IN), lambda i: (0, 0)),
        ],
        out_specs=pl.BlockSpec((_EBLK, D_IN), lambda i: (i, 0)),
        out_shape=jax.ShapeDtypeStruct((N_EDGES, D_IN), jnp.float32),
    )(edge_scalars, edge_attr, fc_w1, fc_w2)


_NBLK = 2000


def _node_pre_body(x_ref, attr_ref, lin1_ref, scw_ref, nf_ref, nsc_ref):
    x = x_ref[...]
    s = attr_ref[...] * (1.0 / np.sqrt(D_IN))
    nf_ref[...] = jnp.dot(x, lin1_ref[...], preferred_element_type=jnp.float32) * s
    nsc_ref[...] = jnp.dot(x, scw_ref[...], preferred_element_type=jnp.float32) * s


def _node_pre(node_input, node_attr, lin1, scw):
    grid = (N_NODES // _NBLK,)
    return pl.pallas_call(
        _node_pre_body,
        grid=grid,
        in_specs=[
            pl.BlockSpec((_NBLK, D_IN), lambda i: (i, 0)),
            pl.BlockSpec((_NBLK, 1), lambda i: (i, 0)),
            pl.BlockSpec((D_IN, D_IN), lambda i: (0, 0)),
            pl.BlockSpec((D_IN, D_IN), lambda i: (0, 0)),
        ],
        out_specs=[
            pl.BlockSpec((_NBLK, D_IN), lambda i: (i, 0)),
            pl.BlockSpec((_NBLK, D_IN), lambda i: (i, 0)),
        ],
        out_shape=[
            jax.ShapeDtypeStruct((N_NODES, D_IN), jnp.float32),
            jax.ShapeDtypeStruct((N_NODES, D_IN), jnp.float32),
        ],
    )(node_input, node_attr, lin1, scw)


_NWORK = 32
_EPW = N_EDGES // _NWORK
_CB = 80
_NCH = _EPW // _CB
_PH0 = 62
_PH1 = _NCH - _PH0
_SRCBUF = _PH1 * _CB
_SLAB = 624
_TAIL = N_NODES - 16 * _SLAB


def _sc_body(nf_hbm, wea_hbm, src_hbm, dst_hbm, out0, out1,
             srcall_v, dst0_v, dst1_v, rows0_v, rows1_v, w0_v, w1_v,
             acc_sh, gsem0, gsem1, wsem0, wsem1, dsem0, dsem1):
    c = lax.axis_index("c")
    s = lax.axis_index("s")
    wid = s * 2 + c

    dst_b = (dst0_v, dst1_v)
    rows_b = (rows0_v, rows1_v)
    w_b = (w0_v, w1_v)
    gsem_b = (gsem0, gsem1)
    wsem_b = (wsem0, wsem1)
    dsem_b = (dsem0, dsem1)

    zero16 = jnp.zeros((16,), jnp.float32)

    def zbody(i, carry):
        rows0_v[i // 8, pl.ds((i % 8) * 16, 16)] = zero16
        return carry

    lax.fori_loop(0, _CB * 8, zbody, 0, unroll=8)

    slab = s * _SLAB

    def zslab(k, carry):
        pltpu.sync_copy(rows0_v, acc_sh.at[pl.ds(slab + k * _CB, _CB), :])
        return carry

    lax.fori_loop(0, _SLAB // _CB, zslab, 0)
    pltpu.sync_copy(rows0_v.at[pl.ds(0, _SLAB - (_SLAB // _CB) * _CB), :],
                    acc_sh.at[pl.ds(slab + (_SLAB // _CB) * _CB,
                                    _SLAB - (_SLAB // _CB) * _CB), :])

    @pl.when(s == 0)
    def _():
        pltpu.sync_copy(rows0_v.at[pl.ds(0, _TAIL), :],
                        acc_sh.at[pl.ds(16 * _SLAB, _TAIL), :])

    base0 = wid * _EPW

    def issue(j, j0, b):
        base = base0 + j * _CB
        pltpu.async_copy(dst_hbm.at[pl.ds(base, _CB)], dst_b[b], dsem_b[b])
        pltpu.async_copy(wea_hbm.at[pl.ds(base, _CB), :], w_b[b], wsem_b[b])
        pltpu.async_copy(nf_hbm.at[srcall_v.at[pl.ds((j - j0) * _CB, _CB)]],
                         rows_b[b], gsem_b[b])

    def wait(j, j0, b):
        base = base0 + j * _CB
        pltpu.make_async_copy(dst_hbm.at[pl.ds(base, _CB)],
                              dst_b[b], dsem_b[b]).wait()
        pltpu.make_async_copy(wea_hbm.at[pl.ds(base, _CB), :],
                              w_b[b], wsem_b[b]).wait()
        pltpu.make_async_copy(nf_hbm.at[srcall_v.at[pl.ds((j - j0) * _CB, _CB)]],
                              rows_b[b], gsem_b[b]).wait()

    def mul_scatter(j, b):
        rows_v = rows_b[b]
        w_v = w_b[b]

        def mul(i, cc):
            r = i // 8
            col = (i % 8) * 16
            rows_v[r, pl.ds(col, 16)] = (
                rows_v[r, pl.ds(col, 16)] * w_v[r, pl.ds(col, 16)])
            return cc

        lax.fori_loop(0, _CB * 8, mul, 0, unroll=8)
        pltpu.sync_copy(rows_v, acc_sh.at[dst_b[b]], add=True)

    def run_phase(j0, nch):
        pltpu.sync_copy(src_hbm.at[pl.ds(base0 + j0 * _CB, nch * _CB)],
                        srcall_v.at[pl.ds(0, nch * _CB)])
        issue(j0, j0, 0)

        def pipe(k, carry):
            j = j0 + 2 * k
            issue(j + 1, j0, 1)
            wait(j, j0, 0)
            mul_scatter(j, 0)
            issue(j + 2, j0, 0)
            wait(j + 1, j0, 1)
            mul_scatter(j + 1, 1)
            return carry

        if nch % 2 == 1:
            lax.fori_loop(0, (nch - 1) // 2, pipe, 0)
            wait(j0 + nch - 1, j0, 0)
            mul_scatter(j0 + nch - 1, 0)
        else:
            lax.fori_loop(0, (nch - 2) // 2, pipe, 0)
            issue(j0 + nch - 1, j0, 1)
            wait(j0 + nch - 2, j0, 0)
            mul_scatter(j0 + nch - 2, 0)
            wait(j0 + nch - 1, j0, 1)
            mul_scatter(j0 + nch - 1, 1)

    run_phase(0, _PH0)
    run_phase(_PH0, _PH1)
    plsc.subcore_barrier()

    @pl.when(c == 0)
    def _():
        pltpu.sync_copy(acc_sh.at[pl.ds(slab, _SLAB), :],
                        out0.at[pl.ds(slab, _SLAB), :])

    @pl.when(c == 1)
    def _():
        pltpu.sync_copy(acc_sh.at[pl.ds(slab, _SLAB), :],
                        out1.at[pl.ds(slab, _SLAB), :])

    @pl.when((s == 0) & (c == 0))
    def _():
        pltpu.sync_copy(acc_sh.at[pl.ds(16 * _SLAB, _TAIL), :],
                        out0.at[pl.ds(16 * _SLAB, _TAIL), :])

    @pl.when((s == 0) & (c == 1))
    def _():
        pltpu.sync_copy(acc_sh.at[pl.ds(16 * _SLAB, _TAIL), :],
                        out1.at[pl.ds(16 * _SLAB, _TAIL), :])


def _sc_scatter(nf, wea, src, dst):
    call = functools.partial(
        pl.kernel,
        out_type=(
            jax.ShapeDtypeStruct((N_NODES, D_IN), jnp.float32),
            jax.ShapeDtypeStruct((N_NODES, D_IN), jnp.float32),
        ),
        mesh=plsc.VectorSubcoreMesh(core_axis_name="c", subcore_axis_name="s",
                                    num_cores=2, num_subcores=16),
        scratch_types=[
            pltpu.VMEM((_SRCBUF,), jnp.int32),
            pltpu.VMEM((_CB,), jnp.int32),
            pltpu.VMEM((_CB,), jnp.int32),
            pltpu.VMEM((_CB, D_IN), jnp.float32),
            pltpu.VMEM((_CB, D_IN), jnp.float32),
            pltpu.VMEM((_CB, D_IN), jnp.float32),
            pltpu.VMEM((_CB, D_IN), jnp.float32),
            pltpu.VMEM_SHARED((N_NODES, D_IN), jnp.float32),
            pltpu.SemaphoreType.DMA,
            pltpu.SemaphoreType.DMA,
            pltpu.SemaphoreType.DMA,
            pltpu.SemaphoreType.DMA,
            pltpu.SemaphoreType.DMA,
            pltpu.SemaphoreType.DMA,
        ],
    )(_sc_body)
    return call(nf, wea, src, dst)


def _final_body(a0_ref, a1_ref, attr_ref, lin2_ref, lin3_ref, nsc_ref, out_ref):
    agg = (a0_ref[...] + a1_ref[...]) * (1.0 / np.sqrt(NUM_NEIGHBORS))
    a = attr_ref[...] * (1.0 / np.sqrt(D_IN))
    conv = jnp.dot(agg, lin2_ref[...], preferred_element_type=jnp.float32) * a
    ang = 0.1 * jnp.dot(agg, lin3_ref[...], preferred_element_type=jnp.float32) * a
    out_ref[...] = jnp.cos(ang) * nsc_ref[...] + jnp.sin(ang) * conv


def _final(a0, a1, node_attr, lin2, lin3, nsc):
    grid = (N_NODES // _NBLK,)
    return pl.pallas_call(
        _final_body,
        grid=grid,
        in_specs=[
            pl.BlockSpec((_NBLK, D_IN), lambda i: (i, 0)),
            pl.BlockSpec((_NBLK, D_IN), lambda i: (i, 0)),
            pl.BlockSpec((_NBLK, 1), lambda i: (i, 0)),
            pl.BlockSpec((D_IN, D_IN), lambda i: (0, 0)),
            pl.BlockSpec((D_IN, 1), lambda i: (0, 0)),
            pl.BlockSpec((_NBLK, D_IN), lambda i: (i, 0)),
        ],
        out_specs=pl.BlockSpec((_NBLK, D_IN), lambda i: (i, 0)),
        out_shape=jax.ShapeDtypeStruct((N_NODES, D_IN), jnp.float32),
    )(a0, a1, node_attr, lin2, lin3, nsc)


def kernel(node_input, node_attr, edge_src, edge_dst, edge_attr, edge_scalars,
           sc_w, lin1_w, fc_w1, fc_w2, lin2_w, lin3_w):
    src = edge_src.astype(jnp.int32)
    dst = edge_dst.astype(jnp.int32)
    wea = _edge_weights(edge_scalars, edge_attr, fc_w1, fc_w2)
    nf, nsc = _node_pre(node_input, node_attr, lin1_w[:, 0, :], sc_w[:, 0, :])
    a0, a1 = _sc_scatter(nf, wea, src, dst)
    return _final(a0, a1, node_attr, lin2_w[:, 0, :], lin3_w[:, 0, :], nsc)
